# baseline (device time: 19589 ns/iter reference)
import jax
import jax.numpy as jnp
from jax import lax
from jax.experimental import pallas as pl
from jax.experimental.pallas import tpu as pltpu

N_DEV = 8
B, SQ, SKV, DH = 2, 256, 256, 64
H_LOC = 4
D_MODEL = 512
SEG = SQ // N_DEV
WINDOW = 128


def kernel(x, Wq, K_ext, V_ext, Wo):
    pos = lax.axis_index("i")
    K_loc = lax.dynamic_slice_in_dim(K_ext, pos * H_LOC, H_LOC, axis=2)
    V_loc = lax.dynamic_slice_in_dim(V_ext, pos * H_LOC, H_LOC, axis=2)

    def body(x_ref, wq_ref, k_ref, v_ref, wo_ref, out_ref,
             acc_ref, snd_ref, rs_ref, p1_send, p1_recv, p2_send, p2_recv):
        my = lax.axis_index("i")

        barrier = pltpu.get_barrier_semaphore()
        for d in range(1, N_DEV):
            pl.semaphore_signal(barrier, inc=1,
                                device_id=(lax.rem(my + d, N_DEV),),
                                device_id_type=pl.DeviceIdType.MESH)
        pl.semaphore_wait(barrier, N_DEV - 1)

        qi = lax.broadcasted_iota(jnp.int32, (SQ, SKV), 0)
        ki = lax.broadcasted_iota(jnp.int32, (SQ, SKV), 1)
        win = jnp.abs(qi - ki) <= WINDOW

        p1 = {}
        p2 = {}

        def scatter(b):
            for d in range(1, N_DEV):
                tgt = lax.rem(my + d, N_DEV)
                slot = d - 1
                rdma = pltpu.make_async_remote_copy(
                    src_ref=snd_ref.at[b].at[pl.ds(tgt, 1)],
                    dst_ref=rs_ref.at[b].at[pl.ds(slot, 1)],
                    send_sem=p1_send.at[b, slot],
                    recv_sem=p1_recv.at[b, slot],
                    device_id=(tgt,),
                    device_id_type=pl.DeviceIdType.MESH,
                )
                rdma.start()
                p1[b, slot] = rdma

        def reduce_bcast(b):
            total = acc_ref[b, pl.ds(my, 1)]
            for slot in range(N_DEV - 1):
                p1[b, slot].wait_recv()
                total = total + rs_ref[b, pl.ds(slot, 1)].astype(jnp.float32)
            snd_ref[b, pl.ds(my, 1)] = total.astype(jnp.bfloat16)
            for d in range(1, N_DEV):
                tgt = lax.rem(my + d, N_DEV)
                slot = d - 1
                rdma = pltpu.make_async_remote_copy(
                    src_ref=snd_ref.at[b].at[pl.ds(my, 1)],
                    dst_ref=snd_ref.at[b].at[pl.ds(my, 1)],
                    send_sem=p2_send.at[b, slot],
                    recv_sem=p2_recv.at[b, slot],
                    device_id=(tgt,),
                    device_id_type=pl.DeviceIdType.MESH,
                )
                rdma.start()
                p2[b, slot] = rdma

        for b in range(B):
            qb = jnp.dot(x_ref[b], wq_ref[...],
                         preferred_element_type=jnp.float32)
            ctx = []
            for h in range(H_LOC):
                qh = qb[:, h * DH:(h + 1) * DH]
                kh = k_ref[b, :, h, :]
                vh = v_ref[b, :, h, :]
                s = lax.dot_general(qh, kh, (((1,), (1,)), ((), ())),
                                    preferred_element_type=jnp.float32)
                s = jnp.where(win, s * 0.125, jnp.float32(-1e9))
                m = jnp.max(s, axis=1, keepdims=True)
                w = jnp.exp(s - m)
                w = w / jnp.sum(w, axis=1, keepdims=True)
                ctx.append(jnp.dot(w, vh,
                                   preferred_element_type=jnp.float32))
            ctxb = jnp.concatenate(ctx, axis=1)
            part = jnp.dot(ctxb, wo_ref[...],
                           preferred_element_type=jnp.float32)
            acc_ref[b] = part.reshape(N_DEV, SEG, D_MODEL)
            snd_ref[b] = part.astype(jnp.bfloat16).reshape(
                N_DEV, SEG, D_MODEL)
            scatter(b)

        for b in range(B):
            reduce_bcast(b)

        for b in range(B):
            for slot in range(N_DEV - 1):
                p2[b, slot].wait_recv()
            out_ref[b] = snd_ref[b].astype(jnp.float32).reshape(SQ, D_MODEL)

        for rdma in list(p1.values()) + list(p2.values()):
            rdma.wait_send()

    return pl.pallas_call(
        body,
        out_shape=jax.ShapeDtypeStruct((B, SQ, D_MODEL), jnp.float32),
        in_specs=[pl.BlockSpec(memory_space=pltpu.VMEM)] * 5,
        out_specs=pl.BlockSpec(memory_space=pltpu.VMEM),
        scratch_shapes=[
            pltpu.VMEM((B, N_DEV, SEG, D_MODEL), jnp.float32),
            pltpu.VMEM((B, N_DEV, SEG, D_MODEL), jnp.bfloat16),
            pltpu.VMEM((B, N_DEV - 1, SEG, D_MODEL), jnp.bfloat16),
            pltpu.SemaphoreType.DMA((B, N_DEV - 1)),
            pltpu.SemaphoreType.DMA((B, N_DEV - 1)),
            pltpu.SemaphoreType.DMA((B, N_DEV - 1)),
            pltpu.SemaphoreType.DMA((B, N_DEV - 1)),
        ],
        compiler_params=pltpu.CompilerParams(collective_id=0),
    )(x, Wq, K_loc, V_loc, Wo)


# device time: 18061 ns/iter; 1.0846x vs baseline; 1.0846x over previous
import jax
import jax.numpy as jnp
from jax import lax
from jax.experimental import pallas as pl
from jax.experimental.pallas import tpu as pltpu

N_DEV = 8
B, SQ, SKV, DH = 2, 256, 256, 64
H_LOC = 4
D_MODEL = 512
SEG = SQ // N_DEV
STRIP = 128
WINDOW = 128


def kernel(x, Wq, K_ext, V_ext, Wo):
    pos = lax.axis_index("i")
    K_loc = lax.dynamic_slice_in_dim(K_ext, pos * H_LOC, H_LOC, axis=2)
    V_loc = lax.dynamic_slice_in_dim(V_ext, pos * H_LOC, H_LOC, axis=2)

    def body(x_ref, wq_ref, k_ref, v_ref, wo_ref, out_ref,
             snd_ref, rs_ref, p1_send, p1_recv, p2_send, p2_recv):
        my = lax.axis_index("i")

        barrier = pltpu.get_barrier_semaphore()
        for d in range(1, N_DEV):
            pl.semaphore_signal(barrier, inc=1,
                                device_id=(lax.rem(my + d, N_DEV),),
                                device_id_type=pl.DeviceIdType.MESH)
        pl.semaphore_wait(barrier, N_DEV - 1)

        qi = lax.broadcasted_iota(jnp.int32, (SQ, SKV), 0)
        ki = lax.broadcasted_iota(jnp.int32, (SQ, SKV), 1)
        win = jnp.abs(qi - ki) <= WINDOW
        wq_s = wq_ref[...] * 0.125

        p1 = {}
        for b in range(B):
            for hs in range(SQ // STRIP):
                r0 = hs * STRIP
                qb = jnp.dot(x_ref[b, r0:r0 + STRIP], wq_s,
                             preferred_element_type=jnp.float32)
                ctx = []
                for h in range(H_LOC):
                    qh = qb[:, h * DH:(h + 1) * DH]
                    kh = k_ref[b, :, h, :]
                    vh = v_ref[b, :, h, :]
                    s = lax.dot_general(qh, kh, (((1,), (1,)), ((), ())),
                                        preferred_element_type=jnp.float32)
                    w = jnp.exp(jnp.where(win[r0:r0 + STRIP], s,
                                          jnp.float32(-1e9)))
                    w = w / jnp.sum(w, axis=1, keepdims=True)
                    ctx.append(jnp.dot(w, vh,
                                       preferred_element_type=jnp.float32))
                ctxb = jnp.concatenate(ctx, axis=1)
                part = jnp.dot(ctxb, wo_ref[...],
                               preferred_element_type=jnp.float32)
                seg0 = r0 // SEG
                nseg = STRIP // SEG
                snd_ref[b, pl.ds(seg0, nseg)] = part.astype(
                    jnp.bfloat16).reshape(nseg, SEG, D_MODEL)
                for o in range(seg0, seg0 + nseg):
                    rdma = pltpu.make_async_remote_copy(
                        src_ref=snd_ref.at[b].at[pl.ds(o, 1)],
                        dst_ref=rs_ref.at[b].at[pl.ds(my, 1)],
                        send_sem=p1_send.at[b, o],
                        recv_sem=p1_recv.at[b, my],
                        device_id=(o,),
                        device_id_type=pl.DeviceIdType.MESH,
                    )
                    rdma.start()
                    p1[b, o] = rdma

        p2 = {}
        for b in range(B):
            total = jnp.zeros((1, SEG, D_MODEL), jnp.float32)
            for slot in range(N_DEV):
                wait = pltpu.make_async_remote_copy(
                    src_ref=rs_ref.at[b].at[pl.ds(slot, 1)],
                    dst_ref=rs_ref.at[b].at[pl.ds(slot, 1)],
                    send_sem=p1_send.at[b, 0],
                    recv_sem=p1_recv.at[b, slot],
                    device_id=(my,),
                    device_id_type=pl.DeviceIdType.MESH,
                )
                wait.wait_recv()
                total = total + rs_ref[b, pl.ds(slot, 1)].astype(jnp.float32)
            snd_ref[b, pl.ds(my, 1)] = total.astype(jnp.bfloat16)
            for d in range(1, N_DEV):
                tgt = lax.rem(my + d, N_DEV)
                slot = d - 1
                rdma = pltpu.make_async_remote_copy(
                    src_ref=snd_ref.at[b].at[pl.ds(my, 1)],
                    dst_ref=snd_ref.at[b].at[pl.ds(my, 1)],
                    send_sem=p2_send.at[b, slot],
                    recv_sem=p2_recv.at[b, slot],
                    device_id=(tgt,),
                    device_id_type=pl.DeviceIdType.MESH,
                )
                rdma.start()
                p2[b, slot] = rdma

        for b in range(B):
            for slot in range(N_DEV - 1):
                p2[b, slot].wait_recv()
            out_ref[b] = snd_ref[b].astype(jnp.float32).reshape(SQ, D_MODEL)

        for rdma in list(p1.values()) + list(p2.values()):
            rdma.wait_send()

    return pl.pallas_call(
        body,
        out_shape=jax.ShapeDtypeStruct((B, SQ, D_MODEL), jnp.float32),
        in_specs=[pl.BlockSpec(memory_space=pltpu.VMEM)] * 5,
        out_specs=pl.BlockSpec(memory_space=pltpu.VMEM),
        scratch_shapes=[
            pltpu.VMEM((B, N_DEV, SEG, D_MODEL), jnp.bfloat16),
            pltpu.VMEM((B, N_DEV, SEG, D_MODEL), jnp.bfloat16),
            pltpu.SemaphoreType.DMA((B, N_DEV)),
            pltpu.SemaphoreType.DMA((B, N_DEV)),
            pltpu.SemaphoreType.DMA((B, N_DEV - 1)),
            pltpu.SemaphoreType.DMA((B, N_DEV - 1)),
        ],
        compiler_params=pltpu.CompilerParams(collective_id=0),
    )(x, Wq, K_loc, V_loc, Wo)
